# baseline (speedup 1.0000x reference)
import functools

import numpy as np

import jax
import jax.numpy as jnp
from jax.experimental import pallas as pl
from jax.experimental.pallas import tpu as pltpu
from jax.sharding import Mesh, PartitionSpec as P

_LN_EPS = 1e-5
_N = 1024
_E = 2048
_LAT = 128
_STEPS = 15
_OUT = 3
_ND_PAD = 16
_ED_PAD = 8
_VMEM_LIMIT = 60 * 1024 * 1024


def _bf(x):
    return x.astype(jnp.bfloat16)


def _dot(a, b):
    return jnp.dot(a, b, preferred_element_type=jnp.float32)


def _ln(x, gamma, beta):
    mean = jnp.mean(x, axis=-1, keepdims=True)
    diff = x - mean
    var = jnp.mean(diff * diff, axis=-1, keepdims=True)
    return diff * jax.lax.rsqrt(var + _LN_EPS) * gamma + beta


def _epd_kernel(nf_ref, ef_ref, s_col_ref, r_col_ref, r_row_ref,
                enw_ref, env_ref, eew_ref, eev_ref,
                ewt_ref, evt_ref, nwt_ref, nvt_ref, dw_ref, dv_ref,
                out_ref, ps_ref, pr_ref, prt_ref, node_buf, edge_buf):
    n, e, lp = _N, _E, _LAT

    iota_en = jax.lax.broadcasted_iota(jnp.int32, (e, n), 1)
    ps_ref[...] = (jnp.broadcast_to(s_col_ref[0], (e, n)) == iota_en
                   ).astype(jnp.bfloat16)
    pr_ref[...] = (jnp.broadcast_to(r_col_ref[0], (e, n)) == iota_en
                   ).astype(jnp.bfloat16)
    iota_ne = jax.lax.broadcasted_iota(jnp.int32, (n, e), 0)
    prt_ref[...] = (jnp.broadcast_to(r_row_ref[0], (n, e)) == iota_ne
                    ).astype(jnp.bfloat16)

    def enc(x, w_ref, v_ref, k0):
        h = _dot(_bf(x), _bf(w_ref[0:k0, :])) + v_ref[0:1, :]
        h = jnp.maximum(h, 0.0)
        h = _dot(_bf(h), _bf(w_ref[k0:k0 + lp, :])) + v_ref[1:2, :]
        h = jnp.maximum(h, 0.0)
        h = _dot(_bf(h), _bf(w_ref[k0 + lp:k0 + 2 * lp, :])) + v_ref[2:3, :]
        return _ln(h, v_ref[3:4, :], v_ref[4:5, :])

    node_buf[...] = enc(nf_ref[0], enw_ref, env_ref, _ND_PAD)
    edge_buf[...] = enc(ef_ref[0], eew_ref, eev_ref, _ED_PAD)

    def block(s, carry):
        ew = ewt_ref[s]
        ev = evt_ref[s]
        nw = nwt_ref[s]
        nv = nvt_ref[s]
        node = node_buf[...]
        edge = edge_buf[...]
        node_b = _bf(node)
        s_term = _dot(ps_ref[...], _bf(_dot(node_b, ew[0:lp, :])))
        r_term = _dot(pr_ref[...], _bf(_dot(node_b, ew[lp:2 * lp, :])))
        x = (s_term + r_term
             + _dot(_bf(edge), ew[2 * lp:3 * lp, :]) + ev[0:1, :])
        x = jnp.maximum(x, 0.0)
        x = _dot(_bf(x), ew[3 * lp:4 * lp, :]) + ev[1:2, :]
        x = jnp.maximum(x, 0.0)
        x = _dot(_bf(x), ew[4 * lp:5 * lp, :]) + ev[2:3, :]
        x = _ln(x, ev[3:4, :], ev[4:5, :])
        edge_buf[...] = edge + x

        agg = _dot(prt_ref[...], _bf(x))
        y = (_dot(node_b, nw[0:lp, :])
             + _dot(_bf(agg), nw[lp:2 * lp, :]) + nv[0:1, :])
        y = jnp.maximum(y, 0.0)
        y = _dot(_bf(y), nw[2 * lp:3 * lp, :]) + nv[1:2, :]
        y = jnp.maximum(y, 0.0)
        y = _dot(_bf(y), nw[3 * lp:4 * lp, :]) + nv[2:3, :]
        y = _ln(y, nv[3:4, :], nv[4:5, :])
        node_buf[...] = node + y
        return carry

    jax.lax.fori_loop(0, _STEPS, block, 0)

    d = node_buf[...]
    d = _dot(_bf(d), _bf(dw_ref[0:lp, :])) + dv_ref[0:1, :]
    d = jnp.maximum(d, 0.0)
    d = _dot(_bf(d), _bf(dw_ref[lp:2 * lp, :])) + dv_ref[1:2, :]
    d = jnp.maximum(d, 0.0)
    out_ref[0] = _dot(_bf(d), _bf(dw_ref[2 * lp:3 * lp, :])) + dv_ref[2:3, :]


def kernel(node_features, edge_features, senders, receivers,
           enc_node_w, enc_node_v, enc_edge_w, enc_edge_v, dec_w, dec_v,
           b0_ew, b0_ev, b0_nw, b0_nv, b1_ew, b1_ev, b1_nw, b1_nv,
           b2_ew, b2_ev, b2_nw, b2_nv, b3_ew, b3_ev, b3_nw, b3_nv,
           b4_ew, b4_ev, b4_nw, b4_nv, b5_ew, b5_ev, b5_nw, b5_nv,
           b6_ew, b6_ev, b6_nw, b6_nv, b7_ew, b7_ev, b7_nw, b7_nv,
           b8_ew, b8_ev, b8_nw, b8_nv, b9_ew, b9_ev, b9_nw, b9_nv,
           b10_ew, b10_ev, b10_nw, b10_nv, b11_ew, b11_ev, b11_nw, b11_nv,
           b12_ew, b12_ev, b12_nw, b12_nv, b13_ew, b13_ev, b13_nw, b13_nv,
           b14_ew, b14_ev, b14_nw, b14_nv):
    b, n, nd = node_features.shape
    _, e, ed = edge_features.shape
    lp = _LAT

    block_args = [
        (b0_ew, b0_ev, b0_nw, b0_nv), (b1_ew, b1_ev, b1_nw, b1_nv),
        (b2_ew, b2_ev, b2_nw, b2_nv), (b3_ew, b3_ev, b3_nw, b3_nv),
        (b4_ew, b4_ev, b4_nw, b4_nv), (b5_ew, b5_ev, b5_nw, b5_nv),
        (b6_ew, b6_ev, b6_nw, b6_nv), (b7_ew, b7_ev, b7_nw, b7_nv),
        (b8_ew, b8_ev, b8_nw, b8_nv), (b9_ew, b9_ev, b9_nw, b9_nv),
        (b10_ew, b10_ev, b10_nw, b10_nv), (b11_ew, b11_ev, b11_nw, b11_nv),
        (b12_ew, b12_ev, b12_nw, b12_nv), (b13_ew, b13_ev, b13_nw, b13_nv),
        (b14_ew, b14_ev, b14_nw, b14_nv),
    ]
    ewt = jnp.stack([_bf(ew) for (ew, _, _, _) in block_args])
    evt = jnp.stack([ev for (_, ev, _, _) in block_args])
    nwt = jnp.stack([_bf(nw) for (_, _, nw, _) in block_args])
    nvt = jnp.stack([nv for (_, _, _, nv) in block_args])

    def fwd(node_f, edge_f, snds, rcvs, enw, env, eew, eev, dw, dv,
            ewts, evts, nwts, nvts):
        bl = node_f.shape[0]
        nf = jnp.pad(node_f, ((0, 0), (0, 0), (0, _ND_PAD - nd)))
        ef = jnp.pad(edge_f, ((0, 0), (0, 0), (0, _ED_PAD - ed)))
        s_col = snds.reshape(bl, e, 1)
        r_col = rcvs.reshape(bl, e, 1)
        r_row = rcvs.reshape(bl, 1, e)
        return pl.pallas_call(
            _epd_kernel,
            out_shape=jax.ShapeDtypeStruct((bl, n, lp), jnp.float32),
            grid_spec=pltpu.PrefetchScalarGridSpec(
                num_scalar_prefetch=0,
                grid=(bl,),
                in_specs=[
                    pl.BlockSpec((1, n, _ND_PAD), lambda i: (i, 0, 0)),
                    pl.BlockSpec((1, e, _ED_PAD), lambda i: (i, 0, 0)),
                    pl.BlockSpec((1, e, 1), lambda i: (i, 0, 0)),
                    pl.BlockSpec((1, e, 1), lambda i: (i, 0, 0)),
                    pl.BlockSpec((1, 1, e), lambda i: (i, 0, 0)),
                    pl.BlockSpec(enw.shape, lambda i: (0, 0)),
                    pl.BlockSpec(env.shape, lambda i: (0, 0)),
                    pl.BlockSpec(eew.shape, lambda i: (0, 0)),
                    pl.BlockSpec(eev.shape, lambda i: (0, 0)),
                    pl.BlockSpec(ewts.shape, lambda i: (0, 0, 0)),
                    pl.BlockSpec(evts.shape, lambda i: (0, 0, 0)),
                    pl.BlockSpec(nwts.shape, lambda i: (0, 0, 0)),
                    pl.BlockSpec(nvts.shape, lambda i: (0, 0, 0)),
                    pl.BlockSpec(dw.shape, lambda i: (0, 0)),
                    pl.BlockSpec(dv.shape, lambda i: (0, 0)),
                ],
                out_specs=pl.BlockSpec((1, n, lp), lambda i: (i, 0, 0)),
                scratch_shapes=[
                    pltpu.VMEM((e, n), jnp.bfloat16),
                    pltpu.VMEM((e, n), jnp.bfloat16),
                    pltpu.VMEM((n, e), jnp.bfloat16),
                    pltpu.VMEM((n, lp), jnp.float32),
                    pltpu.VMEM((e, lp), jnp.float32),
                ],
            ),
            compiler_params=pltpu.CompilerParams(
                dimension_semantics=("parallel",),
                vmem_limit_bytes=_VMEM_LIMIT),
        )(nf, ef, s_col, r_col, r_row, _bf(enw), env, _bf(eew), eev,
          ewts, evts, nwts, nvts, _bf(dw), dv)

    def fwd_out(node_f, edge_f, snds, rcvs, *ws):
        return fwd(node_f, edge_f, snds, rcvs, *ws)[:, :, :_OUT]

    args = (node_features, edge_features, senders, receivers,
            enc_node_w, enc_node_v, enc_edge_w, enc_edge_v, dec_w, dec_v,
            ewt, evt, nwt, nvt)
    devs = jax.devices()
    if len(devs) > 1 and b % len(devs) == 0:
        mesh = Mesh(np.array(devs), ("b",))
        return jax.shard_map(
            fwd_out, mesh=mesh,
            in_specs=(P("b"),) * 4 + (P(),) * 10,
            out_specs=P("b"), check_vma=False)(*args)
    return fwd_out(*args)
